# baseline (speedup 1.0000x reference)
import functools

import jax
import jax.numpy as jnp
from jax import lax
from jax.experimental import pallas as pl
from jax.experimental.pallas import tpu as pltpu
from jax.experimental.pallas import tpu_sc as plsc

B = 16384
D = 256
V = 1000
VP = 1024
BS = 8192
BT = B - BS

_info = plsc.get_sparse_core_info()
NC = _info.num_cores
NS = _info.num_subcores
NW = NC * NS
BPW = BS // NW
C = 128
NCHUNK = BPW // C
NBUF = min(3, NCHUNK)

_mesh = plsc.VectorSubcoreMesh(core_axis_name="c", subcore_axis_name="s")


@functools.partial(
    pl.kernel,
    mesh=_mesh,
    out_type=jax.ShapeDtypeStruct((BS, D), jnp.float32),
    scratch_types=(
        [pltpu.VMEM((NCHUNK, C), jnp.int32)]
        + [pltpu.VMEM((C, D), jnp.float32)] * NBUF
        + [pltpu.SemaphoreType.DMA] * (2 * NBUF)
    ),
)
def _sc_gather(table_hbm, idx_hbm, out_hbm, idx_v, *rest):
    bufs = list(rest[:NBUF])
    gsems = list(rest[NBUF:2 * NBUF])
    osems = list(rest[2 * NBUF:])
    wid = lax.axis_index("s") * NC + lax.axis_index("c")
    base = wid * BPW
    pltpu.sync_copy(idx_hbm.at[wid], idx_v)

    def gather(j):
        k = j % NBUF
        return pltpu.async_copy(table_hbm.at[idx_v.at[j]], bufs[k], gsems[k])

    def writeback(j):
        k = j % NBUF
        return pltpu.async_copy(
            bufs[k], out_hbm.at[pl.ds(base + j * C, C)], osems[k])

    gathers = [gather(j) for j in range(NBUF)]
    writes = [None] * NCHUNK
    for j in range(NCHUNK):
        gathers[j % NBUF].wait()
        writes[j] = writeback(j)
        nxt = j + NBUF
        if nxt < NCHUNK:
            writes[nxt - NBUF].wait()
            gathers[nxt % NBUF] = gather(nxt)
    for j in range(max(0, NCHUNK - NBUF), NCHUNK):
        writes[j].wait()


M = 512


def _tc_body(idx_ref, hi_ref, lo_ref, out_ref):
    ids = idx_ref[0, 0, :]
    iota = lax.broadcasted_iota(jnp.int32, (M, VP), 1)
    oh = (ids[:, None] == iota).astype(jnp.bfloat16)
    dn = (((1,), (0,)), ((), ()))
    acc = lax.dot_general(oh, hi_ref[...], dn,
                          preferred_element_type=jnp.float32)
    acc += lax.dot_general(oh, lo_ref[...], dn,
                           preferred_element_type=jnp.float32)
    out_ref[...] = acc


_tc_onehot = pl.pallas_call(
    _tc_body,
    grid=(BT // M,),
    in_specs=[
        pl.BlockSpec((1, 1, M), lambda i: (i, 0, 0)),
        pl.BlockSpec((VP, D), lambda i: (0, 0)),
        pl.BlockSpec((VP, D), lambda i: (0, 0)),
    ],
    out_specs=pl.BlockSpec((M, D), lambda i: (i, 0)),
    out_shape=jax.ShapeDtypeStruct((BT, D), jnp.float32),
)


def kernel(emotion_ids, emb_e_weight):
    ids = emotion_ids.astype(jnp.int32)
    idx_sc = ids[:BS].reshape(NW, NCHUNK, C)
    idx_tc = ids[BS:].reshape(BT // M, 1, M)
    tab_p = jnp.pad(emb_e_weight, ((0, VP - V), (0, 0)))
    hi = tab_p.astype(jnp.bfloat16)
    lo = (tab_p - hi.astype(jnp.float32)).astype(jnp.bfloat16)
    out_sc = _sc_gather(emb_e_weight, idx_sc)
    out_tc = _tc_onehot(idx_tc, hi, lo)
    return jnp.concatenate([out_sc, out_tc], axis=0)


# speedup vs baseline: 1.3779x; 1.3779x over previous
import functools

import jax
import jax.numpy as jnp
from jax import lax
from jax.experimental import pallas as pl
from jax.experimental.pallas import tpu as pltpu
from jax.experimental.pallas import tpu_sc as plsc

B = 16384
D = 256
V = 1000

_info = plsc.get_sparse_core_info()
NC = _info.num_cores
NS = _info.num_subcores
NW = NC * NS
BPW = B // NW
C = 128
NCHUNK = BPW // C

_mesh = plsc.VectorSubcoreMesh(core_axis_name="c", subcore_axis_name="s")


NBUF = 3


@functools.partial(
    pl.kernel,
    mesh=_mesh,
    out_type=jax.ShapeDtypeStruct((B, D), jnp.float32),
    scratch_types=[
        pltpu.VMEM((NCHUNK, C), jnp.int32),
        pltpu.VMEM((C, D), jnp.float32),
        pltpu.VMEM((C, D), jnp.float32),
        pltpu.VMEM((C, D), jnp.float32),
        pltpu.SemaphoreType.DMA,
        pltpu.SemaphoreType.DMA,
        pltpu.SemaphoreType.DMA,
        pltpu.SemaphoreType.DMA,
        pltpu.SemaphoreType.DMA,
        pltpu.SemaphoreType.DMA,
    ],
)
def _gather_kernel(table_hbm, idx_hbm, out_hbm, idx_v,
                   b0, b1, b2, sg0, sg1, sg2, so0, so1, so2):
    wid = lax.axis_index("s") * NC + lax.axis_index("c")
    base = wid * BPW
    bufs = [b0, b1, b2]
    gsems = [sg0, sg1, sg2]
    osems = [so0, so1, so2]
    pltpu.sync_copy(idx_hbm.at[wid], idx_v)

    def gather(j):
        k = j % NBUF
        return pltpu.async_copy(table_hbm.at[idx_v.at[j]], bufs[k], gsems[k])

    def writeback(j):
        k = j % NBUF
        return pltpu.async_copy(
            bufs[k], out_hbm.at[pl.ds(base + j * C, C)], osems[k])

    gathers = [gather(j) for j in range(NBUF)]
    writes = [None] * NCHUNK
    for j in range(NCHUNK):
        gathers[j % NBUF].wait()
        writes[j] = writeback(j)
        nxt = j + NBUF
        if nxt < NCHUNK:
            writes[nxt - NBUF].wait()
            gathers[nxt % NBUF] = gather(nxt)
    for j in range(max(0, NCHUNK - NBUF), NCHUNK):
        writes[j].wait()


def kernel(emotion_ids, emb_e_weight):
    idx = emotion_ids.astype(jnp.int32).reshape(NW, NCHUNK, C)
    return _gather_kernel(emb_e_weight, idx)
